# baseline (speedup 1.0000x reference)
import functools

import jax
import jax.numpy as jnp
from jax.experimental import pallas as pl
from jax.experimental.pallas import tpu as pltpu

BN_EPS = 1e-5
LEAKY_SLOPE = 0.2
VMEM_LIMIT_BYTES = 32 * 1024 * 1024
_DOT_DT = jnp.bfloat16
_MID_DT = jnp.bfloat16


def _lrelu(z):
    return jnp.maximum(z, LEAKY_SLOPE * z)


def _dot(w, a):
    return jnp.dot(w, a.astype(_DOT_DT), preferred_element_type=jnp.float32)


def _init_acc(j, sum_ref, ssq_ref):
    @pl.when(j == 0)
    def _():
        sum_ref[...] = jnp.zeros_like(sum_ref)
        ssq_ref[...] = jnp.zeros_like(ssq_ref)


def _acc(y, sum_ref, ssq_ref):
    sum_ref[...] += jnp.sum(y, axis=1, keepdims=True)
    ssq_ref[...] += jnp.sum(y * y, axis=1, keepdims=True)


def _stats1_kernel(x_ref, w1_ref, sum_ref, ssq_ref, *, bn):
    _init_acc(pl.program_id(1), sum_ref, ssq_ref)
    w1 = w1_ref[...]
    for i in range(bn):
        _acc(_dot(w1, x_ref[i]), sum_ref, ssq_ref)


def _stats2_kernel(a_ref, w1f_ref, t1_ref, w2_ref, sum_ref, ssq_ref, *, bn):
    _init_acc(pl.program_id(1), sum_ref, ssq_ref)
    w1f = w1f_ref[...]
    w2 = w2_ref[...]
    for i in range(bn):
        z1 = _lrelu(_dot(w1f, a_ref[i]) + t1_ref[...])
        _acc(_dot(w2, z1), sum_ref, ssq_ref)


def _final_fused_kernel(a_ref, w1f_ref, t1_ref, w2f_ref, t2_ref, wn_ref,
                        o_ref, sum_ref, ssq_ref, *, bn):
    _init_acc(pl.program_id(1), sum_ref, ssq_ref)
    w1f = w1f_ref[...]
    w2f = w2f_ref[...]
    wn = wn_ref[...]
    for i in range(bn):
        z1 = _lrelu(_dot(w1f, a_ref[i]) + t1_ref[...])
        z2 = _lrelu(_dot(w2f, z1) + t2_ref[...])
        o_ref[i] = z2.astype(o_ref.dtype)
        _acc(_dot(wn, z2), sum_ref, ssq_ref)


def _final_kernel(a_ref, w1f_ref, t1_ref, w2f_ref, t2_ref, o_ref, *, bn):
    w1f = w1f_ref[...]
    w2f = w2f_ref[...]
    for i in range(bn):
        z1 = _lrelu(_dot(w1f, a_ref[i]) + t1_ref[...])
        o_ref[i] = _lrelu(_dot(w2f, z1) + t2_ref[...])


def _fold_bn(ch_sum, ch_ssq, gamma, beta, m_real):
    mean = ch_sum / m_real
    var = jnp.maximum(ch_ssq / m_real - mean * mean, 0.0)
    scale = gamma * jax.lax.rsqrt(var + BN_EPS)
    shift = beta - mean * scale
    return scale, shift


def kernel(x,
           w1_0, g1_0, b1_0, w2_0, g2_0, b2_0,
           w1_1, g1_1, b1_1, w2_1, g2_1, b2_1,
           w1_2, g1_2, b1_2, w2_2, g2_2, b2_2):
    params = [((w1_0, g1_0, b1_0), (w2_0, g2_0, b2_0)),
              ((w1_1, g1_1, b1_1), (w2_1, g2_1, b2_1)),
              ((w1_2, g1_2, b1_2), (w2_2, g2_2, b2_2))]

    n, c_in, h, w = x.shape
    hw = h * w
    m_real = n * hw
    a = x.reshape(n, c_in, hw)

    bn = 2
    num_cores = 2 if n >= 2 * bn else 1
    steps = -(-n // bn)
    spc = steps // num_cores
    grid = (num_cores, spc)

    cp_acc = pltpu.CompilerParams(dimension_semantics=("parallel", "arbitrary"),
                                  vmem_limit_bytes=VMEM_LIMIT_BYTES)
    cp_par = pltpu.CompilerParams(dimension_semantics=("parallel", "parallel"),
                                  vmem_limit_bytes=VMEM_LIMIT_BYTES)

    def act_spec(ch):
        return pl.BlockSpec((bn, ch, hw),
                            lambda core, j: (core * spc + j, 0, 0))

    def full_spec(shape):
        nd = len(shape)
        return pl.BlockSpec(tuple(shape), lambda core, j: (0,) * nd)

    def acc_spec(ch):
        return pl.BlockSpec((None, ch, 1), lambda core, j: (core, 0, 0))

    def acc_shape(ch):
        return jax.ShapeDtypeStruct((num_cores, ch, 1), jnp.float32)

    w1b0 = w1_0.astype(_DOT_DT)
    c1_0 = w1b0.shape[0]
    sum1, ssq1 = pl.pallas_call(
        functools.partial(_stats1_kernel, bn=bn),
        grid=grid,
        in_specs=[act_spec(c_in), full_spec(w1b0.shape)],
        out_specs=(acc_spec(c1_0), acc_spec(c1_0)),
        out_shape=(acc_shape(c1_0), acc_shape(c1_0)),
        compiler_params=cp_acc,
    )(a, w1b0)
    sum1, ssq1 = sum1.sum(axis=0), ssq1.sum(axis=0)

    nblocks = len(params)
    for bi, ((w1, g1, b1), (w2, g2, b2)) in enumerate(params):
        cin = a.shape[1]
        c2 = w2.shape[0]

        s1, t1 = _fold_bn(sum1, ssq1, g1, b1, m_real)
        w1f = (w1 * s1).astype(_DOT_DT)
        w2b = w2.astype(_DOT_DT)

        sum2, ssq2 = pl.pallas_call(
            functools.partial(_stats2_kernel, bn=bn),
            grid=grid,
            in_specs=[act_spec(cin), full_spec(w1f.shape), full_spec(t1.shape),
                      full_spec(w2b.shape)],
            out_specs=(acc_spec(c2), acc_spec(c2)),
            out_shape=(acc_shape(c2), acc_shape(c2)),
            compiler_params=cp_acc,
        )(a, w1f, t1, w2b)
        sum2, ssq2 = sum2.sum(axis=0), ssq2.sum(axis=0)
        s2, t2 = _fold_bn(sum2, ssq2, g2, b2, m_real)
        w2f = (w2 * s2).astype(_DOT_DT)

        if bi + 1 < nblocks:
            wn = params[bi + 1][0][0].astype(_DOT_DT)
            c1n = wn.shape[0]
            a, sum1, ssq1 = pl.pallas_call(
                functools.partial(_final_fused_kernel, bn=bn),
                grid=grid,
                in_specs=[act_spec(cin), full_spec(w1f.shape),
                          full_spec(t1.shape), full_spec(w2f.shape),
                          full_spec(t2.shape), full_spec(wn.shape)],
                out_specs=(act_spec(c2), acc_spec(c1n), acc_spec(c1n)),
                out_shape=(jax.ShapeDtypeStruct((n, c2, hw), _MID_DT),
                           acc_shape(c1n), acc_shape(c1n)),
                compiler_params=cp_acc,
            )(a, w1f, t1, w2f, t2, wn)
            sum1, ssq1 = sum1.sum(axis=0), ssq1.sum(axis=0)
        else:
            a = pl.pallas_call(
                functools.partial(_final_kernel, bn=bn),
                grid=grid,
                in_specs=[act_spec(cin), full_spec(w1f.shape),
                          full_spec(t1.shape), full_spec(w2f.shape),
                          full_spec(t2.shape)],
                out_specs=act_spec(c2),
                out_shape=jax.ShapeDtypeStruct((n, c2, hw), jnp.float32),
                compiler_params=cp_par,
            )(a, w1f, t1, w2f, t2)

    return a.reshape(n, a.shape[1], h, w)


# speedup vs baseline: 1.3426x; 1.3426x over previous
import functools

import jax
import jax.numpy as jnp
from jax.experimental import pallas as pl
from jax.experimental.pallas import tpu as pltpu

BN_EPS = 1e-5
LEAKY_SLOPE = 0.2
VMEM_LIMIT_BYTES = 32 * 1024 * 1024
_DOT_DT = jnp.bfloat16
_MID_DT = jnp.bfloat16


def _lrelu(z):
    return jnp.maximum(z, LEAKY_SLOPE * z)


def _dot(w, a):
    return jnp.dot(w, a.astype(_DOT_DT), preferred_element_type=jnp.float32)


def _init_acc(j, sum_ref, ssq_ref):
    @pl.when(j == 0)
    def _():
        sum_ref[...] = jnp.zeros_like(sum_ref)
        ssq_ref[...] = jnp.zeros_like(ssq_ref)


def _acc(y, sum_ref, ssq_ref):
    sum_ref[...] += jnp.sum(y, axis=1, keepdims=True)
    ssq_ref[...] += jnp.sum(y * y, axis=1, keepdims=True)


def _stats1_kernel(x_ref, w1_ref, xc_ref, sum_ref, ssq_ref, *, bn):
    _init_acc(pl.program_id(1), sum_ref, ssq_ref)
    w1 = w1_ref[...]
    ch = x_ref.shape[1]
    m = x_ref.shape[2] * x_ref.shape[3]
    for i in range(bn):
        xi = x_ref[i].reshape(ch, m)
        xc_ref[i] = xi.astype(xc_ref.dtype)
        _acc(_dot(w1, xi), sum_ref, ssq_ref)


def _stats2_kernel(a_ref, w1f_ref, t1_ref, w2_ref, sum_ref, ssq_ref, *, bn):
    _init_acc(pl.program_id(1), sum_ref, ssq_ref)
    w1f = w1f_ref[...]
    w2 = w2_ref[...]
    for i in range(bn):
        z1 = _lrelu(_dot(w1f, a_ref[i]) + t1_ref[...])
        _acc(_dot(w2, z1), sum_ref, ssq_ref)


def _final_fused_kernel(a_ref, w1f_ref, t1_ref, w2f_ref, t2_ref, wn_ref,
                        o_ref, sum_ref, ssq_ref, *, bn):
    _init_acc(pl.program_id(1), sum_ref, ssq_ref)
    w1f = w1f_ref[...]
    w2f = w2f_ref[...]
    wn = wn_ref[...]
    for i in range(bn):
        z1 = _lrelu(_dot(w1f, a_ref[i]) + t1_ref[...])
        z2 = _lrelu(_dot(w2f, z1) + t2_ref[...])
        o_ref[i] = z2.astype(o_ref.dtype)
        _acc(_dot(wn, z2), sum_ref, ssq_ref)


def _final_kernel(a_ref, w1f_ref, t1_ref, w2f_ref, t2_ref, o_ref, *, bn):
    w1f = w1f_ref[...]
    w2f = w2f_ref[...]
    ch, hh, ww = o_ref.shape[1], o_ref.shape[2], o_ref.shape[3]
    for i in range(bn):
        z1 = _lrelu(_dot(w1f, a_ref[i]) + t1_ref[...])
        z2 = _lrelu(_dot(w2f, z1) + t2_ref[...])
        o_ref[i] = z2.reshape(ch, hh, ww)


def _fold_bn(ch_sum, ch_ssq, gamma, beta, m_real):
    mean = ch_sum / m_real
    var = jnp.maximum(ch_ssq / m_real - mean * mean, 0.0)
    scale = gamma * jax.lax.rsqrt(var + BN_EPS)
    shift = beta - mean * scale
    return scale, shift


def kernel(x,
           w1_0, g1_0, b1_0, w2_0, g2_0, b2_0,
           w1_1, g1_1, b1_1, w2_1, g2_1, b2_1,
           w1_2, g1_2, b1_2, w2_2, g2_2, b2_2):
    params = [((w1_0, g1_0, b1_0), (w2_0, g2_0, b2_0)),
              ((w1_1, g1_1, b1_1), (w2_1, g2_1, b2_1)),
              ((w1_2, g1_2, b1_2), (w2_2, g2_2, b2_2))]

    n, c_in, h, w = x.shape
    hw = h * w
    m_real = n * hw

    bn = 4
    num_cores = 2 if n >= 2 * bn else 1
    steps = -(-n // bn)
    spc = steps // num_cores
    grid = (num_cores, spc)

    cp_acc = pltpu.CompilerParams(dimension_semantics=("parallel", "arbitrary"),
                                  vmem_limit_bytes=VMEM_LIMIT_BYTES)
    cp_par = pltpu.CompilerParams(dimension_semantics=("parallel", "parallel"),
                                  vmem_limit_bytes=VMEM_LIMIT_BYTES)

    def act_spec(ch):
        return pl.BlockSpec((bn, ch, hw),
                            lambda core, j: (core * spc + j, 0, 0))

    def act4_spec(ch):
        return pl.BlockSpec((bn, ch, h, w),
                            lambda core, j: (core * spc + j, 0, 0, 0))

    def full_spec(shape):
        nd = len(shape)
        return pl.BlockSpec(tuple(shape), lambda core, j: (0,) * nd)

    def acc_spec(ch):
        return pl.BlockSpec((None, ch, 1), lambda core, j: (core, 0, 0))

    def acc_shape(ch):
        return jax.ShapeDtypeStruct((num_cores, ch, 1), jnp.float32)

    w1b0 = w1_0.astype(_DOT_DT)
    c1_0 = w1b0.shape[0]
    a, sum1, ssq1 = pl.pallas_call(
        functools.partial(_stats1_kernel, bn=bn),
        grid=grid,
        in_specs=[act4_spec(c_in), full_spec(w1b0.shape)],
        out_specs=(act_spec(c_in), acc_spec(c1_0), acc_spec(c1_0)),
        out_shape=(jax.ShapeDtypeStruct((n, c_in, hw), _MID_DT),
                   acc_shape(c1_0), acc_shape(c1_0)),
        compiler_params=cp_acc,
    )(x, w1b0)
    sum1, ssq1 = sum1.sum(axis=0), ssq1.sum(axis=0)

    nblocks = len(params)
    for bi, ((w1, g1, b1), (w2, g2, b2)) in enumerate(params):
        cin = a.shape[1]
        c2 = w2.shape[0]

        s1, t1 = _fold_bn(sum1, ssq1, g1, b1, m_real)
        w1f = (w1 * s1).astype(_DOT_DT)
        w2b = w2.astype(_DOT_DT)

        sum2, ssq2 = pl.pallas_call(
            functools.partial(_stats2_kernel, bn=bn),
            grid=grid,
            in_specs=[act_spec(cin), full_spec(w1f.shape), full_spec(t1.shape),
                      full_spec(w2b.shape)],
            out_specs=(acc_spec(c2), acc_spec(c2)),
            out_shape=(acc_shape(c2), acc_shape(c2)),
            compiler_params=cp_acc,
        )(a, w1f, t1, w2b)
        sum2, ssq2 = sum2.sum(axis=0), ssq2.sum(axis=0)
        s2, t2 = _fold_bn(sum2, ssq2, g2, b2, m_real)
        w2f = (w2 * s2).astype(_DOT_DT)

        if bi + 1 < nblocks:
            wn = params[bi + 1][0][0].astype(_DOT_DT)
            c1n = wn.shape[0]
            a, sum1, ssq1 = pl.pallas_call(
                functools.partial(_final_fused_kernel, bn=bn),
                grid=grid,
                in_specs=[act_spec(cin), full_spec(w1f.shape),
                          full_spec(t1.shape), full_spec(w2f.shape),
                          full_spec(t2.shape), full_spec(wn.shape)],
                out_specs=(act_spec(c2), acc_spec(c1n), acc_spec(c1n)),
                out_shape=(jax.ShapeDtypeStruct((n, c2, hw), _MID_DT),
                           acc_shape(c1n), acc_shape(c1n)),
                compiler_params=cp_acc,
            )(a, w1f, t1, w2f, t2, wn)
            sum1, ssq1 = sum1.sum(axis=0), ssq1.sum(axis=0)
        else:
            a = pl.pallas_call(
                functools.partial(_final_kernel, bn=bn),
                grid=grid,
                in_specs=[act_spec(cin), full_spec(w1f.shape),
                          full_spec(t1.shape), full_spec(w2f.shape),
                          full_spec(t2.shape)],
                out_specs=act4_spec(c2),
                out_shape=jax.ShapeDtypeStruct((n, c2, h, w), jnp.float32),
                compiler_params=cp_par,
            )(a, w1f, t1, w2f, t2)

    return a


# speedup vs baseline: 1.5815x; 1.1779x over previous
import functools

import jax
import jax.numpy as jnp
from jax.experimental import pallas as pl
from jax.experimental.pallas import tpu as pltpu

BN_EPS = 1e-5
LEAKY_SLOPE = 0.2
_DOT_DT = jnp.bfloat16
_MID_DT = jnp.bfloat16


def _lrelu(z):
    return jnp.maximum(z, LEAKY_SLOPE * z)


def _dot(w, a):
    return jnp.dot(w, a.astype(_DOT_DT), preferred_element_type=jnp.float32)


def _init_acc(refs):
    @pl.when(pl.program_id(0) == 0)
    def _():
        for r in refs:
            r[...] = jnp.zeros_like(r)


def _acc(y, sum_ref, ssq_ref):
    sum_ref[...] += jnp.sum(y, axis=1, keepdims=True)
    ssq_ref[...] += jnp.sum(y * y, axis=1, keepdims=True)


def _fold(sum_ref, ssq_ref, g_ref, b_ref, m_real):
    mean = sum_ref[...] / m_real
    var = jnp.maximum(ssq_ref[...] / m_real - mean * mean, 0.0)
    scale = g_ref[...] * jax.lax.rsqrt(var + BN_EPS)
    shift = b_ref[...] - mean * scale
    return scale, shift


def _stats1_kernel(x_ref, w1_ref, xc_ref, sum_ref, ssq_ref, *, bn):
    _init_acc((sum_ref, ssq_ref))
    w1 = w1_ref[...].astype(_DOT_DT)
    ch = x_ref.shape[1]
    m = x_ref.shape[2] * x_ref.shape[3]
    for i in range(bn):
        xi = x_ref[i].reshape(ch, m)
        xc_ref[i] = xi.astype(xc_ref.dtype)
        _acc(_dot(w1, xi), sum_ref, ssq_ref)


def _stats2_kernel(a_ref, w1_ref, g1_ref, b1_ref, sum1_ref, ssq1_ref,
                   w2_ref, sum_ref, ssq_ref, *, bn, m_real):
    _init_acc((sum_ref, ssq_ref))
    s1, t1 = _fold(sum1_ref, ssq1_ref, g1_ref, b1_ref, m_real)
    w1f = (w1_ref[...] * s1).astype(_DOT_DT)
    w2 = w2_ref[...].astype(_DOT_DT)
    for i in range(bn):
        z1 = _lrelu(_dot(w1f, a_ref[i]) + t1)
        _acc(_dot(w2, z1), sum_ref, ssq_ref)


def _final_fused_kernel(a_ref, w1_ref, g1_ref, b1_ref, sum1_ref, ssq1_ref,
                        w2_ref, g2_ref, b2_ref, sum2_ref, ssq2_ref, wn_ref,
                        o_ref, sum_ref, ssq_ref, *, bn, m_real):
    _init_acc((sum_ref, ssq_ref))
    s1, t1 = _fold(sum1_ref, ssq1_ref, g1_ref, b1_ref, m_real)
    w1f = (w1_ref[...] * s1).astype(_DOT_DT)
    s2, t2 = _fold(sum2_ref, ssq2_ref, g2_ref, b2_ref, m_real)
    w2f = (w2_ref[...] * s2).astype(_DOT_DT)
    wn = wn_ref[...].astype(_DOT_DT)
    for i in range(bn):
        z1 = _lrelu(_dot(w1f, a_ref[i]) + t1)
        z2 = _lrelu(_dot(w2f, z1) + t2)
        o_ref[i] = z2.astype(o_ref.dtype)
        _acc(_dot(wn, z2), sum_ref, ssq_ref)


def _final_kernel(a_ref, w1_ref, g1_ref, b1_ref, sum1_ref, ssq1_ref,
                  w2_ref, g2_ref, b2_ref, sum2_ref, ssq2_ref,
                  o_ref, *, bn, m_real):
    s1, t1 = _fold(sum1_ref, ssq1_ref, g1_ref, b1_ref, m_real)
    w1f = (w1_ref[...] * s1).astype(_DOT_DT)
    s2, t2 = _fold(sum2_ref, ssq2_ref, g2_ref, b2_ref, m_real)
    w2f = (w2_ref[...] * s2).astype(_DOT_DT)
    ch, hh, ww = o_ref.shape[1], o_ref.shape[2], o_ref.shape[3]
    for i in range(bn):
        z1 = _lrelu(_dot(w1f, a_ref[i]) + t1)
        z2 = _lrelu(_dot(w2f, z1) + t2)
        o_ref[i] = z2.reshape(ch, hh, ww)


def kernel(x,
           w1_0, g1_0, b1_0, w2_0, g2_0, b2_0,
           w1_1, g1_1, b1_1, w2_1, g2_1, b2_1,
           w1_2, g1_2, b1_2, w2_2, g2_2, b2_2):
    params = [((w1_0, g1_0, b1_0), (w2_0, g2_0, b2_0)),
              ((w1_1, g1_1, b1_1), (w2_1, g2_1, b2_1)),
              ((w1_2, g1_2, b1_2), (w2_2, g2_2, b2_2))]

    n, c_in, h, w = x.shape
    hw = h * w
    m_real = n * hw

    bn = 4
    steps = -(-n // bn)
    grid = (steps,)

    cp = pltpu.CompilerParams(dimension_semantics=("arbitrary",),
                              vmem_limit_bytes=24 * 1024 * 1024)

    def act_spec(ch):
        return pl.BlockSpec((bn, ch, hw), lambda j: (j, 0, 0))

    def act4_spec(ch):
        return pl.BlockSpec((bn, ch, h, w), lambda j: (j, 0, 0, 0))

    def full_spec(shape):
        nd = len(shape)
        return pl.BlockSpec(tuple(shape), lambda j: (0,) * nd)

    def acc_spec(ch):
        return pl.BlockSpec((ch, 1), lambda j: (0, 0))

    def acc_shape(ch):
        return jax.ShapeDtypeStruct((ch, 1), jnp.float32)

    c1 = w1_0.shape[0]
    c2 = w2_0.shape[0]

    a, sum1, ssq1 = pl.pallas_call(
        functools.partial(_stats1_kernel, bn=bn),
        grid=grid,
        in_specs=[act4_spec(c_in), full_spec(w1_0.shape)],
        out_specs=(act_spec(c_in), acc_spec(c1), acc_spec(c1)),
        out_shape=(jax.ShapeDtypeStruct((n, c_in, hw), _MID_DT),
                   acc_shape(c1), acc_shape(c1)),
        compiler_params=cp,
    )(x, w1_0)

    nblocks = len(params)
    for bi, ((w1, g1, b1), (w2, g2, b2)) in enumerate(params):
        cin = a.shape[1]
        c1b, c2b = w1.shape[0], w2.shape[0]

        sum2, ssq2 = pl.pallas_call(
            functools.partial(_stats2_kernel, bn=bn, m_real=m_real),
            grid=grid,
            in_specs=[act_spec(cin), full_spec(w1.shape), full_spec(g1.shape),
                      full_spec(b1.shape), acc_spec(c1b), acc_spec(c1b),
                      full_spec(w2.shape)],
            out_specs=(acc_spec(c2b), acc_spec(c2b)),
            out_shape=(acc_shape(c2b), acc_shape(c2b)),
            compiler_params=cp,
        )(a, w1, g1, b1, sum1, ssq1, w2)

        common_ins = [a, w1, g1, b1, sum1, ssq1, w2, g2, b2, sum2, ssq2]
        common_specs = [act_spec(cin), full_spec(w1.shape),
                        full_spec(g1.shape), full_spec(b1.shape),
                        acc_spec(c1b), acc_spec(c1b), full_spec(w2.shape),
                        full_spec(g2.shape), full_spec(b2.shape),
                        acc_spec(c2b), acc_spec(c2b)]

        if bi + 1 < nblocks:
            wn = params[bi + 1][0][0]
            c1n = wn.shape[0]
            a, sum1, ssq1 = pl.pallas_call(
                functools.partial(_final_fused_kernel, bn=bn, m_real=m_real),
                grid=grid,
                in_specs=common_specs + [full_spec(wn.shape)],
                out_specs=(act_spec(c2b), acc_spec(c1n), acc_spec(c1n)),
                out_shape=(jax.ShapeDtypeStruct((n, c2b, hw), _MID_DT),
                           acc_shape(c1n), acc_shape(c1n)),
                compiler_params=cp,
            )(*common_ins, wn)
        else:
            a = pl.pallas_call(
                functools.partial(_final_kernel, bn=bn, m_real=m_real),
                grid=grid,
                in_specs=common_specs,
                out_specs=act4_spec(c2b),
                out_shape=jax.ShapeDtypeStruct((n, c2b, h, w), jnp.float32),
                compiler_params=cp,
            )(*common_ins)

    return a
